# Initial kernel scaffold; baseline (speedup 1.0000x reference)
import jax
import jax.numpy as jnp
from jax.experimental import pallas as pl


def kernel(styles, const, b0_A, b0_b2, b0_w_taps, b0_wsq, b0_noise_strength, b1_A, b1_b2, b1_w_taps, b1_wsq, b1_noise_strength, b2_A, b2_b2, b2_w_taps, b2_wsq, b2_noise_strength, b3_A, b3_b2, b3_w_taps, b3_wsq, b3_noise_strength, b4_A, b4_b2, b4_w_taps, b4_wsq, b4_noise_strength, b5_A, b5_b2, b5_w_taps, b5_wsq, b5_noise_strength, b6_A, b6_b2, b6_w_taps, b6_wsq, b6_noise_strength, b7_A, b7_b2, b7_w_taps, b7_wsq, b7_noise_strength, b8_A, b8_b2, b8_w_taps, b8_wsq, b8_noise_strength, rgb_A, rgb_b2, rgb_w_mat, noise0, noise1, noise2, noise3, noise4, noise5, noise6, noise7, noise8):
    raise NotImplementedError("write your pallas kernel here")


# speedup vs baseline: 1.1198x; 1.1198x over previous
import functools

import jax
import jax.numpy as jnp
from jax import lax
from jax.experimental import pallas as pl
from jax.experimental.pallas import tpu as pltpu

_EPS = 1e-8
_UPSAMPLE = (False, True, False, True, False, True, False, True, False)
_VMEM_LIMIT = int(min(96 * 1024 * 1024, (3 * (64 * 1024 * 1024)) // 4))


def _genblock_kernel(style_ref, x_ref, a_ref, b2_ref, wt_ref, wsq_ref, noise_ref,
                     ns_ref, out_ref, *, taps, guard, n_rows, bt):
    cout = out_ref.shape[-1]
    s = jnp.dot(style_ref[0], a_ref[...],
                preferred_element_type=jnp.float32) + b2_ref[...]
    d = lax.rsqrt(jnp.dot(s * s, wsq_ref[...],
                          preferred_element_type=jnp.float32) + _EPS)
    ns = ns_ref[0]
    for k in range(bt):
        xm = (x_ref[k] * s[k:k + 1, :]).astype(jnp.bfloat16)
        acc = jnp.zeros((n_rows, cout), jnp.float32)
        for t, delta in enumerate(taps):
            start = guard + delta
            acc = acc + jnp.dot(xm[start:start + n_rows, :], wt_ref[t],
                                preferred_element_type=jnp.float32)
        nz = jnp.transpose(noise_ref[0, k:k + 1, :])
        y = acc * d[k:k + 1, :] + ns * nz
        out_ref[k] = jnp.where(y >= 0.0, y, 0.2 * y)


def _rgb_kernel(style_ref, x_ref, a_ref, b2_ref, w_ref, out_ref, *, bt):
    s = jnp.dot(style_ref[0], a_ref[...],
                preferred_element_type=jnp.float32) + b2_ref[...]
    for k in range(bt):
        y = jnp.dot(x_ref[k] * s[k:k + 1, :], w_ref[...],
                    preferred_element_type=jnp.float32)
        out_ref[k] = jnp.tanh(y)


def _genblock_call(x, style, A, b2, w_taps, wsq, ns, noise):
    B, H, W, cin = x.shape
    cout = w_taps.shape[-1]
    nz = style.shape[1]
    hp, wp = H + 2, W + 2
    n_rows = hp * wp
    guard = wp + 1
    bt = 16 if H <= 16 else (8 if H <= 32 else 2)

    xp = jnp.pad(x, ((0, 0), (1, 1), (1, 1), (0, 0))).reshape(B, n_rows, cin)
    xp = jnp.pad(xp, ((0, 0), (guard, guard), (0, 0)))
    nflat = jnp.pad(noise, ((0, 0), (1, 1), (1, 1), (0, 0))).reshape(B // bt, bt, n_rows)
    sty3 = style.reshape(B // bt, bt, nz)
    wt = w_taps.astype(jnp.bfloat16)

    taps = tuple((ky - 1) * wp + (kx - 1) for ky in range(3) for kx in range(3))
    kern = functools.partial(_genblock_kernel, taps=taps, guard=guard,
                             n_rows=n_rows, bt=bt)
    out = pl.pallas_call(
        kern,
        out_shape=jax.ShapeDtypeStruct((B, n_rows, cout), jnp.float32),
        grid=(B // bt,),
        in_specs=[
            pl.BlockSpec((1, bt, nz), lambda i: (i, 0, 0)),
            pl.BlockSpec((bt, n_rows + 2 * guard, cin), lambda i: (i, 0, 0)),
            pl.BlockSpec((nz, cin), lambda i: (0, 0)),
            pl.BlockSpec((1, cin), lambda i: (0, 0)),
            pl.BlockSpec((9, cin, cout), lambda i: (0, 0, 0)),
            pl.BlockSpec((cin, cout), lambda i: (0, 0)),
            pl.BlockSpec((1, bt, n_rows), lambda i: (i, 0, 0)),
            pl.BlockSpec(memory_space=pltpu.MemorySpace.SMEM),
        ],
        out_specs=pl.BlockSpec((bt, n_rows, cout), lambda i: (i, 0, 0)),
        compiler_params=pltpu.CompilerParams(
            dimension_semantics=("parallel",),
            vmem_limit_bytes=_VMEM_LIMIT,
        ),
    )(sty3, xp, A, b2, wt, wsq, nflat, ns)
    return out.reshape(B, hp, wp, cout)[:, 1:1 + H, 1:1 + W, :]


def _rgb_call(x, style, A, b2, w_mat):
    B, H, W, cin = x.shape
    nz = style.shape[1]
    n_rows = H * W
    bt = 2
    xflat = x.reshape(B, n_rows, cin)
    sty3 = style.reshape(B // bt, bt, nz)
    out = pl.pallas_call(
        functools.partial(_rgb_kernel, bt=bt),
        out_shape=jax.ShapeDtypeStruct((B, n_rows, 3), jnp.float32),
        grid=(B // bt,),
        in_specs=[
            pl.BlockSpec((1, bt, nz), lambda i: (i, 0, 0)),
            pl.BlockSpec((bt, n_rows, cin), lambda i: (i, 0, 0)),
            pl.BlockSpec((nz, cin), lambda i: (0, 0)),
            pl.BlockSpec((1, cin), lambda i: (0, 0)),
            pl.BlockSpec((cin, 3), lambda i: (0, 0)),
        ],
        out_specs=pl.BlockSpec((bt, n_rows, 3), lambda i: (i, 0, 0)),
        compiler_params=pltpu.CompilerParams(
            dimension_semantics=("parallel",),
            vmem_limit_bytes=_VMEM_LIMIT,
        ),
    )(sty3, xflat, A, b2, w_mat)
    return out.reshape(B, H, W, 3)


def _upsample2x_axis(x, axis):
    n = x.shape[axis]
    idx = jnp.arange(n)
    x_prev = jnp.take(x, jnp.maximum(idx - 1, 0), axis=axis)
    x_next = jnp.take(x, jnp.minimum(idx + 1, n - 1), axis=axis)
    even = 0.25 * x_prev + 0.75 * x
    odd = 0.75 * x + 0.25 * x_next
    y = jnp.stack([even, odd], axis=axis + 1)
    shape = list(x.shape)
    shape[axis] = 2 * n
    return y.reshape(shape)


def kernel(styles, const,
           b0_A, b0_b2, b0_w_taps, b0_wsq, b0_noise_strength,
           b1_A, b1_b2, b1_w_taps, b1_wsq, b1_noise_strength,
           b2_A, b2_b2, b2_w_taps, b2_wsq, b2_noise_strength,
           b3_A, b3_b2, b3_w_taps, b3_wsq, b3_noise_strength,
           b4_A, b4_b2, b4_w_taps, b4_wsq, b4_noise_strength,
           b5_A, b5_b2, b5_w_taps, b5_wsq, b5_noise_strength,
           b6_A, b6_b2, b6_w_taps, b6_wsq, b6_noise_strength,
           b7_A, b7_b2, b7_w_taps, b7_wsq, b7_noise_strength,
           b8_A, b8_b2, b8_w_taps, b8_wsq, b8_noise_strength,
           rgb_A, rgb_b2, rgb_w_mat,
           noise0, noise1, noise2, noise3, noise4,
           noise5, noise6, noise7, noise8):
    blocks = [
        (b0_A, b0_b2, b0_w_taps, b0_wsq, b0_noise_strength),
        (b1_A, b1_b2, b1_w_taps, b1_wsq, b1_noise_strength),
        (b2_A, b2_b2, b2_w_taps, b2_wsq, b2_noise_strength),
        (b3_A, b3_b2, b3_w_taps, b3_wsq, b3_noise_strength),
        (b4_A, b4_b2, b4_w_taps, b4_wsq, b4_noise_strength),
        (b5_A, b5_b2, b5_w_taps, b5_wsq, b5_noise_strength),
        (b6_A, b6_b2, b6_w_taps, b6_wsq, b6_noise_strength),
        (b7_A, b7_b2, b7_w_taps, b7_wsq, b7_noise_strength),
        (b8_A, b8_b2, b8_w_taps, b8_wsq, b8_noise_strength),
    ]
    noises = [noise0, noise1, noise2, noise3, noise4,
              noise5, noise6, noise7, noise8]
    B = styles.shape[0]
    nf = const.shape[-1]
    x = jnp.broadcast_to(const[None], (B, 4, 4, nf))
    for j, (A, b2, w_taps, wsq, ns) in enumerate(blocks):
        if _UPSAMPLE[j]:
            x = _upsample2x_axis(_upsample2x_axis(x, 1), 2)
        x = _genblock_call(x, styles[:, j, :], A, b2, w_taps, wsq, ns, noises[j])
    return _rgb_call(x, styles[:, -1, :], rgb_A, rgb_b2, rgb_w_mat)


# speedup vs baseline: 1.3156x; 1.1748x over previous
import functools

import jax
import jax.numpy as jnp
from jax import lax
from jax.experimental import pallas as pl
from jax.experimental.pallas import tpu as pltpu

_EPS = 1e-8
_UPSAMPLE = (False, True, False, True, False, True, False, True, False)
_VMEM_LIMIT = int(min(96 * 1024 * 1024, (3 * (64 * 1024 * 1024)) // 4))


def _guard(wp):
    return ((wp + 2 + 7) // 8) * 8


def _genblock_kernel(style_ref, x_ref, a_ref, b2_ref, wt_ref, wsq_ref, noise_ref,
                     ns_ref, out_ref, *, taps, g, n_rows, hp, wp, bt):
    cout = out_ref.shape[-1]
    s = jnp.dot(style_ref[0], a_ref[...],
                preferred_element_type=jnp.float32) + b2_ref[...]
    d = lax.rsqrt(jnp.dot(s * s, wsq_ref[...],
                          preferred_element_type=jnp.float32) + _EPS)
    ns = ns_ref[0]
    r = lax.broadcasted_iota(jnp.int32, (n_rows, 1), 0)
    yc = r // wp
    xc = r - yc * wp
    interior = (yc >= 1) & (yc <= hp - 2) & (xc >= 1) & (xc <= wp - 2)
    zg = jnp.zeros((g, cout), jnp.float32)
    for k in range(bt):
        xm = (x_ref[k] * s[k:k + 1, :]).astype(jnp.bfloat16)
        acc = jnp.zeros((n_rows, cout), jnp.float32)
        for t, delta in enumerate(taps):
            start = g + delta
            acc = acc + jnp.dot(xm[start:start + n_rows, :], wt_ref[t],
                                preferred_element_type=jnp.float32)
        nz = jnp.transpose(noise_ref[0, k:k + 1, :])
        y = acc * d[k:k + 1, :] + ns * nz
        y = jnp.where(y >= 0.0, y, 0.2 * y)
        y = jnp.where(interior, y, 0.0)
        out_ref[k] = jnp.concatenate([zg, y, zg], axis=0)


def _rgb_kernel(style_ref, x_ref, a_ref, b2_ref, w_ref, out_ref, *, bt):
    s = jnp.dot(style_ref[0], a_ref[...],
                preferred_element_type=jnp.float32) + b2_ref[...]
    for k in range(bt):
        y = jnp.dot(x_ref[k] * s[k:k + 1, :], w_ref[...],
                    preferred_element_type=jnp.float32)
        out_ref[k] = jnp.tanh(y)


def _genblock_call(x, H, W, style, A, b2, w_taps, wsq, ns, noise):
    B, T, cin = x.shape
    cout = w_taps.shape[-1]
    nz = style.shape[1]
    hp, wp = H + 2, W + 2
    n_rows = hp * wp
    g = _guard(wp)
    bt = 16 if H <= 16 else (8 if H <= 32 else 2)

    nflat = jnp.pad(noise, ((0, 0), (1, 1), (1, 1), (0, 0)))
    nflat = nflat.reshape(B // bt, bt, n_rows)
    sty3 = style.reshape(B // bt, bt, nz)
    wt = w_taps.astype(jnp.bfloat16)

    taps = tuple((ky - 1) * wp + (kx - 1) for ky in range(3) for kx in range(3))
    kern = functools.partial(_genblock_kernel, taps=taps, g=g, n_rows=n_rows,
                             hp=hp, wp=wp, bt=bt)
    return pl.pallas_call(
        kern,
        out_shape=jax.ShapeDtypeStruct((B, T, cout), jnp.float32),
        grid=(B // bt,),
        in_specs=[
            pl.BlockSpec((1, bt, nz), lambda i: (i, 0, 0)),
            pl.BlockSpec((bt, T, cin), lambda i: (i, 0, 0)),
            pl.BlockSpec((nz, cin), lambda i: (0, 0)),
            pl.BlockSpec((1, cin), lambda i: (0, 0)),
            pl.BlockSpec((9, cin, cout), lambda i: (0, 0, 0)),
            pl.BlockSpec((cin, cout), lambda i: (0, 0)),
            pl.BlockSpec((1, bt, n_rows), lambda i: (i, 0, 0)),
            pl.BlockSpec(memory_space=pltpu.MemorySpace.SMEM),
        ],
        out_specs=pl.BlockSpec((bt, T, cout), lambda i: (i, 0, 0)),
        compiler_params=pltpu.CompilerParams(
            dimension_semantics=("parallel",),
            vmem_limit_bytes=_VMEM_LIMIT,
        ),
    )(sty3, x, A, b2, wt, wsq, nflat, ns)


def _rgb_call(x, H, W, style, A, b2, w_mat):
    B, T, cin = x.shape
    nz = style.shape[1]
    hp, wp = H + 2, W + 2
    g = _guard(wp)
    bt = 2
    sty3 = style.reshape(B // bt, bt, nz)
    out = pl.pallas_call(
        functools.partial(_rgb_kernel, bt=bt),
        out_shape=jax.ShapeDtypeStruct((B, T, 3), jnp.float32),
        grid=(B // bt,),
        in_specs=[
            pl.BlockSpec((1, bt, nz), lambda i: (i, 0, 0)),
            pl.BlockSpec((bt, T, cin), lambda i: (i, 0, 0)),
            pl.BlockSpec((nz, cin), lambda i: (0, 0)),
            pl.BlockSpec((1, cin), lambda i: (0, 0)),
            pl.BlockSpec((cin, 3), lambda i: (0, 0)),
        ],
        out_specs=pl.BlockSpec((bt, T, 3), lambda i: (i, 0, 0)),
        compiler_params=pltpu.CompilerParams(
            dimension_semantics=("parallel",),
            vmem_limit_bytes=_VMEM_LIMIT,
        ),
    )(sty3, x, A, b2, w_mat)
    out = out[:, g:g + hp * wp, :].reshape(B, hp, wp, 3)
    return out[:, 1:1 + H, 1:1 + W, :]


def _upsample_padded(x, H, W):
    B, T, C = x.shape
    hp, wp = H + 2, W + 2
    g = _guard(wp)
    xi = x[:, g:g + hp * wp, :].reshape(B, hp, wp, C)[:, 1:1 + H, 1:1 + W, :]

    def up_axis(v, axis):
        n = v.shape[axis]
        first = lax.slice_in_dim(v, 0, 1, axis=axis)
        last = lax.slice_in_dim(v, n - 1, n, axis=axis)
        prev = jnp.concatenate([first, lax.slice_in_dim(v, 0, n - 1, axis=axis)], axis)
        nxt = jnp.concatenate([lax.slice_in_dim(v, 1, n, axis=axis), last], axis)
        even = 0.25 * prev + 0.75 * v
        odd = 0.75 * v + 0.25 * nxt
        y = jnp.stack([even, odd], axis=axis + 1)
        shape = list(v.shape)
        shape[axis] = 2 * n
        return y.reshape(shape)

    xb = up_axis(up_axis(xi, 1), 2)
    H2, W2 = 2 * H, 2 * W
    g2 = _guard(W2 + 2)
    xb = jnp.pad(xb, ((0, 0), (1, 1), (1, 1), (0, 0)))
    xb = xb.reshape(B, (H2 + 2) * (W2 + 2), C)
    return jnp.pad(xb, ((0, 0), (g2, g2), (0, 0)))


def kernel(styles, const,
           b0_A, b0_b2, b0_w_taps, b0_wsq, b0_noise_strength,
           b1_A, b1_b2, b1_w_taps, b1_wsq, b1_noise_strength,
           b2_A, b2_b2, b2_w_taps, b2_wsq, b2_noise_strength,
           b3_A, b3_b2, b3_w_taps, b3_wsq, b3_noise_strength,
           b4_A, b4_b2, b4_w_taps, b4_wsq, b4_noise_strength,
           b5_A, b5_b2, b5_w_taps, b5_wsq, b5_noise_strength,
           b6_A, b6_b2, b6_w_taps, b6_wsq, b6_noise_strength,
           b7_A, b7_b2, b7_w_taps, b7_wsq, b7_noise_strength,
           b8_A, b8_b2, b8_w_taps, b8_wsq, b8_noise_strength,
           rgb_A, rgb_b2, rgb_w_mat,
           noise0, noise1, noise2, noise3, noise4,
           noise5, noise6, noise7, noise8):
    blocks = [
        (b0_A, b0_b2, b0_w_taps, b0_wsq, b0_noise_strength),
        (b1_A, b1_b2, b1_w_taps, b1_wsq, b1_noise_strength),
        (b2_A, b2_b2, b2_w_taps, b2_wsq, b2_noise_strength),
        (b3_A, b3_b2, b3_w_taps, b3_wsq, b3_noise_strength),
        (b4_A, b4_b2, b4_w_taps, b4_wsq, b4_noise_strength),
        (b5_A, b5_b2, b5_w_taps, b5_wsq, b5_noise_strength),
        (b6_A, b6_b2, b6_w_taps, b6_wsq, b6_noise_strength),
        (b7_A, b7_b2, b7_w_taps, b7_wsq, b7_noise_strength),
        (b8_A, b8_b2, b8_w_taps, b8_wsq, b8_noise_strength),
    ]
    noises = [noise0, noise1, noise2, noise3, noise4,
              noise5, noise6, noise7, noise8]
    B = styles.shape[0]
    nf = const.shape[-1]

    H = W = 4
    g0 = _guard(W + 2)
    cflat = jnp.pad(const, ((1, 1), (1, 1), (0, 0))).reshape((H + 2) * (W + 2), nf)
    cflat = jnp.pad(cflat, ((g0, g0), (0, 0)))
    x = jnp.broadcast_to(cflat[None], (B, cflat.shape[0], nf))

    for j, (A, b2, w_taps, wsq, ns) in enumerate(blocks):
        if _UPSAMPLE[j]:
            x = _upsample_padded(x, H, W)
            H, W = 2 * H, 2 * W
        x = _genblock_call(x, H, W, styles[:, j, :], A, b2, w_taps, wsq, ns,
                           noises[j])
    return _rgb_call(x, H, W, styles[:, -1, :], rgb_A, rgb_b2, rgb_w_mat)


# speedup vs baseline: 1.8997x; 1.4440x over previous
import functools

import jax
import jax.numpy as jnp
from jax import lax
from jax.experimental import pallas as pl
from jax.experimental.pallas import tpu as pltpu

_EPS = 1e-8
_UPSAMPLE = (False, True, False, True, False, True, False, True, False)
_VMEM_LIMIT = int(min(96 * 1024 * 1024, (3 * (64 * 1024 * 1024)) // 4))


def _guard(wp):
    return ((wp + 2 + 7) // 8) * 8


def _kron_eye(w, p):
    if p == 1:
        return w
    a, b = w.shape
    return (jnp.eye(p, dtype=w.dtype)[:, None, :, None] *
            w[None, :, None, :]).reshape(p * a, p * b)


def _genblock_kernel(style_ref, x_ref, a_ref, b2_ref, wt_ref, wsq_ref, noise_ref,
                     ns_ref, out_ref, *, taps, g, n_rows, hp, wp, p_in, rg, no,
                     cout):
    cin_p = x_ref.shape[-1]
    cout_p = p_in * cout
    s = jnp.dot(style_ref[0], a_ref[...],
                preferred_element_type=jnp.float32) + b2_ref[...]
    d = lax.rsqrt(jnp.dot(s * s, wsq_ref[...],
                          preferred_element_type=jnp.float32) + _EPS)
    ns = ns_ref[0]
    r = lax.broadcasted_iota(jnp.int32, (n_rows, 1), 0)
    yc = r // wp
    xc = r - yc * wp
    interior = (yc >= 1) & (yc <= hp - 2) & (xc >= 1) & (xc <= wp - 2)
    zg = jnp.zeros((g, rg * cout_p), jnp.float32)
    if p_in > 1:
        io_l = lax.broadcasted_iota(jnp.int32, (p_in, cout_p), 1)
        io_p = lax.broadcasted_iota(jnp.int32, (p_in, cout_p), 0)
        expand = (io_l // cout == io_p).astype(jnp.float32)
    for o in range(no):
        ys = []
        for q in range(rg):
            gi = o * rg + q
            xm = (x_ref[gi] * s[gi:gi + 1, :]).astype(jnp.bfloat16)
            acc = jnp.zeros((n_rows, cout_p), jnp.float32)
            for t, delta in enumerate(taps):
                start = g + delta
                acc = acc + jnp.dot(xm[start:start + n_rows, :], wt_ref[t],
                                    preferred_element_type=jnp.float32)
            if p_in > 1:
                nz = jnp.dot(jnp.transpose(noise_ref[0, gi]), expand,
                             preferred_element_type=jnp.float32)
            else:
                nz = jnp.transpose(noise_ref[0, gi])
            y = acc * d[gi:gi + 1, :] + ns * nz
            y = jnp.where(y >= 0.0, y, 0.2 * y)
            ys.append(jnp.where(interior, y, 0.0))
        y_all = ys[0] if rg == 1 else jnp.concatenate(ys, axis=1)
        out_ref[o] = jnp.concatenate([zg, y_all, zg], axis=0)


def _rgb_kernel(style_ref, x_ref, a_ref, b2_ref, w_ref, out_ref, *, no):
    s = jnp.dot(style_ref[0], a_ref[...],
                preferred_element_type=jnp.float32) + b2_ref[...]
    for k in range(no):
        y = jnp.dot(x_ref[k] * s[k:k + 1, :], w_ref[...],
                    preferred_element_type=jnp.float32)
        out_ref[k] = jnp.tanh(y)


def _genblock_call(x, H, W, style, A, b2, w_taps, wsq, ns, noise,
                   p_in=1, rg=1, no=None):
    Bg, T, cin_p = x.shape
    cin = cin_p // p_in
    cout = w_taps.shape[-1]
    nz = A.shape[0]
    hp, wp = H + 2, W + 2
    n_rows = hp * wp
    g = _guard(wp)
    if no is None:
        no = 16 if H <= 16 else (8 if H <= 32 else 2)
    grid = Bg // (no * rg)

    B = Bg * p_in
    nflat = jnp.pad(noise, ((0, 0), (1, 1), (1, 1), (0, 0)))
    nflat = nflat.reshape(grid, no * rg, p_in, n_rows)
    sty_p = style.reshape(grid, no * rg, p_in * nz)
    wt = jnp.stack([_kron_eye(w_taps[t], p_in) for t in range(9)], 0)
    wt = wt.astype(jnp.bfloat16)
    A_p = _kron_eye(A, p_in)
    b2_p = jnp.tile(b2, (1, p_in))
    wsq_p = _kron_eye(wsq, p_in)

    taps = tuple((ky - 1) * wp + (kx - 1) for ky in range(3) for kx in range(3))
    kern = functools.partial(_genblock_kernel, taps=taps, g=g, n_rows=n_rows,
                             hp=hp, wp=wp, p_in=p_in, rg=rg, no=no, cout=cout)
    return pl.pallas_call(
        kern,
        out_shape=jax.ShapeDtypeStruct((Bg // rg, T, rg * p_in * cout),
                                       jnp.float32),
        grid=(grid,),
        in_specs=[
            pl.BlockSpec((1, no * rg, p_in * nz), lambda i: (i, 0, 0)),
            pl.BlockSpec((no * rg, T, cin_p), lambda i: (i, 0, 0)),
            pl.BlockSpec((p_in * nz, cin_p), lambda i: (0, 0)),
            pl.BlockSpec((1, cin_p), lambda i: (0, 0)),
            pl.BlockSpec((9, cin_p, p_in * cout), lambda i: (0, 0, 0)),
            pl.BlockSpec((cin_p, p_in * cout), lambda i: (0, 0)),
            pl.BlockSpec((1, no * rg, p_in, n_rows), lambda i: (i, 0, 0, 0)),
            pl.BlockSpec(memory_space=pltpu.MemorySpace.SMEM),
        ],
        out_specs=pl.BlockSpec((no, T, rg * p_in * cout), lambda i: (i, 0, 0)),
        compiler_params=pltpu.CompilerParams(
            dimension_semantics=("parallel",),
            vmem_limit_bytes=_VMEM_LIMIT,
        ),
    )(sty_p, x, A_p, b2_p, wt, wsq_p, nflat, ns)


def _rgb_call(x, H, W, style, A, b2, w_mat, p_in):
    Bg, T, cin_p = x.shape
    nz = A.shape[0]
    hp, wp = H + 2, W + 2
    g = _guard(wp)
    no = 2
    B = Bg * p_in
    sty_p = style.reshape(Bg // no, no, p_in * nz)
    A_p = _kron_eye(A, p_in)
    b2_p = jnp.tile(b2, (1, p_in))
    w_p = _kron_eye(w_mat, p_in)
    out = pl.pallas_call(
        functools.partial(_rgb_kernel, no=no),
        out_shape=jax.ShapeDtypeStruct((Bg, T, p_in * 3), jnp.float32),
        grid=(Bg // no,),
        in_specs=[
            pl.BlockSpec((1, no, p_in * nz), lambda i: (i, 0, 0)),
            pl.BlockSpec((no, T, cin_p), lambda i: (i, 0, 0)),
            pl.BlockSpec((p_in * nz, cin_p), lambda i: (0, 0)),
            pl.BlockSpec((1, cin_p), lambda i: (0, 0)),
            pl.BlockSpec((cin_p, p_in * 3), lambda i: (0, 0)),
        ],
        out_specs=pl.BlockSpec((no, T, p_in * 3), lambda i: (i, 0, 0)),
        compiler_params=pltpu.CompilerParams(
            dimension_semantics=("parallel",),
            vmem_limit_bytes=_VMEM_LIMIT,
        ),
    )(sty_p, x, A_p, b2_p, w_p)
    o = out[:, g:g + hp * wp, :].reshape(Bg, hp, wp, p_in, 3)
    o = jnp.transpose(o, (0, 3, 1, 2, 4)).reshape(B, hp, wp, 3)
    return o[:, 1:1 + H, 1:1 + W, :]


def _upsample_padded(x, H, W):
    Bg, T, C = x.shape
    hp, wp = H + 2, W + 2
    g = _guard(wp)
    xi = x[:, g:g + hp * wp, :].reshape(Bg, hp, wp, C)[:, 1:1 + H, 1:1 + W, :]

    def up_axis(v, axis):
        n = v.shape[axis]
        first = lax.slice_in_dim(v, 0, 1, axis=axis)
        last = lax.slice_in_dim(v, n - 1, n, axis=axis)
        prev = jnp.concatenate([first, lax.slice_in_dim(v, 0, n - 1, axis=axis)], axis)
        nxt = jnp.concatenate([lax.slice_in_dim(v, 1, n, axis=axis), last], axis)
        even = 0.25 * prev + 0.75 * v
        odd = 0.75 * v + 0.25 * nxt
        y = jnp.stack([even, odd], axis=axis + 1)
        shape = list(v.shape)
        shape[axis] = 2 * n
        return y.reshape(shape)

    xb = up_axis(up_axis(xi, 1), 2)
    H2, W2 = 2 * H, 2 * W
    g2 = _guard(W2 + 2)
    xb = jnp.pad(xb, ((0, 0), (1, 1), (1, 1), (0, 0)))
    xb = xb.reshape(Bg, (H2 + 2) * (W2 + 2), C)
    return jnp.pad(xb, ((0, 0), (g2, g2), (0, 0)))


def kernel(styles, const,
           b0_A, b0_b2, b0_w_taps, b0_wsq, b0_noise_strength,
           b1_A, b1_b2, b1_w_taps, b1_wsq, b1_noise_strength,
           b2_A, b2_b2, b2_w_taps, b2_wsq, b2_noise_strength,
           b3_A, b3_b2, b3_w_taps, b3_wsq, b3_noise_strength,
           b4_A, b4_b2, b4_w_taps, b4_wsq, b4_noise_strength,
           b5_A, b5_b2, b5_w_taps, b5_wsq, b5_noise_strength,
           b6_A, b6_b2, b6_w_taps, b6_wsq, b6_noise_strength,
           b7_A, b7_b2, b7_w_taps, b7_wsq, b7_noise_strength,
           b8_A, b8_b2, b8_w_taps, b8_wsq, b8_noise_strength,
           rgb_A, rgb_b2, rgb_w_mat,
           noise0, noise1, noise2, noise3, noise4,
           noise5, noise6, noise7, noise8):
    blocks = [
        (b0_A, b0_b2, b0_w_taps, b0_wsq, b0_noise_strength),
        (b1_A, b1_b2, b1_w_taps, b1_wsq, b1_noise_strength),
        (b2_A, b2_b2, b2_w_taps, b2_wsq, b2_noise_strength),
        (b3_A, b3_b2, b3_w_taps, b3_wsq, b3_noise_strength),
        (b4_A, b4_b2, b4_w_taps, b4_wsq, b4_noise_strength),
        (b5_A, b5_b2, b5_w_taps, b5_wsq, b5_noise_strength),
        (b6_A, b6_b2, b6_w_taps, b6_wsq, b6_noise_strength),
        (b7_A, b7_b2, b7_w_taps, b7_wsq, b7_noise_strength),
        (b8_A, b8_b2, b8_w_taps, b8_wsq, b8_noise_strength),
    ]
    noises = [noise0, noise1, noise2, noise3, noise4,
              noise5, noise6, noise7, noise8]
    plan = [
        (1, 1, 16), (1, 1, 16), (1, 1, 16), (1, 1, 16), (1, 1, 16),
        (1, 2, 4),
        (2, 2, 2),
        (4, 1, 1),
        (4, 1, 1),
    ]
    B = styles.shape[0]
    nf = const.shape[-1]

    H = W = 4
    g0 = _guard(W + 2)
    cflat = jnp.pad(const, ((1, 1), (1, 1), (0, 0))).reshape((H + 2) * (W + 2), nf)
    cflat = jnp.pad(cflat, ((g0, g0), (0, 0)))
    x = jnp.broadcast_to(cflat[None], (B, cflat.shape[0], nf))

    for j, (A, b2, w_taps, wsq, ns) in enumerate(blocks):
        if _UPSAMPLE[j]:
            x = _upsample_padded(x, H, W)
            H, W = 2 * H, 2 * W
        p_in, rg, no = plan[j]
        x = _genblock_call(x, H, W, styles[:, j, :], A, b2, w_taps, wsq, ns,
                           noises[j], p_in=p_in, rg=rg, no=no)
    return _rgb_call(x, H, W, styles[:, -1, :], rgb_A, rgb_b2, rgb_w_mat, p_in=4)


# speedup vs baseline: 2.1440x; 1.1286x over previous
import functools

import jax
import jax.numpy as jnp
from jax import lax
from jax.experimental import pallas as pl
from jax.experimental.pallas import tpu as pltpu

_EPS = 1e-8
_UPSAMPLE = (False, True, False, True, False, True, False, True, False)
_VMEM_LIMIT = int(min(96 * 1024 * 1024, (3 * (64 * 1024 * 1024)) // 4))


def _guard(wp):
    return ((wp + 2 + 7) // 8) * 8


def _kron_eye(w, p):
    if p == 1:
        return w
    a, b = w.shape
    return (jnp.eye(p, dtype=w.dtype)[:, None, :, None] *
            w[None, :, None, :]).reshape(p * a, p * b)


def _genblock_kernel(style_ref, x_ref, a_ref, b2_ref, wt_ref, wsq_ref, noise_ref,
                     ns_ref, out_ref, *, taps, g, n_rows, hp, wp, p_in, rg, no,
                     cout):
    cin_p = x_ref.shape[-1]
    cout_p = p_in * cout
    s = jnp.dot(style_ref[0], a_ref[...],
                preferred_element_type=jnp.float32) + b2_ref[...]
    d = lax.rsqrt(jnp.dot(s * s, wsq_ref[...],
                          preferred_element_type=jnp.float32) + _EPS)
    ns = ns_ref[0]
    r = lax.broadcasted_iota(jnp.int32, (n_rows, 1), 0)
    yc = r // wp
    xc = r - yc * wp
    interior = (yc >= 1) & (yc <= hp - 2) & (xc >= 1) & (xc <= wp - 2)
    zg = jnp.zeros((g, rg * cout_p), jnp.float32)
    if p_in > 1:
        io_l = lax.broadcasted_iota(jnp.int32, (p_in, cout_p), 1)
        io_p = lax.broadcasted_iota(jnp.int32, (p_in, cout_p), 0)
        expand = (io_l // cout == io_p).astype(jnp.float32)
    for o in range(no):
        ys = []
        for q in range(rg):
            gi = o * rg + q
            xm = (x_ref[gi] * s[gi:gi + 1, :]).astype(jnp.bfloat16)
            acc = jnp.zeros((n_rows, cout_p), jnp.float32)
            for t, delta in enumerate(taps):
                start = g + delta
                acc = acc + jnp.dot(xm[start:start + n_rows, :], wt_ref[t],
                                    preferred_element_type=jnp.float32)
            if p_in > 1:
                nz = jnp.dot(jnp.transpose(noise_ref[0, gi]), expand,
                             preferred_element_type=jnp.float32)
            else:
                nz = jnp.transpose(noise_ref[0, gi])
            y = acc * d[gi:gi + 1, :] + ns * nz
            y = jnp.where(y >= 0.0, y, 0.2 * y)
            ys.append(jnp.where(interior, y, 0.0))
        y_all = ys[0] if rg == 1 else jnp.concatenate(ys, axis=1)
        out_ref[o] = jnp.concatenate([zg, y_all, zg], axis=0)


def _rgb_kernel(style_ref, x_ref, a_ref, b2_ref, w_ref, out_ref, *, no):
    s = jnp.dot(style_ref[0], a_ref[...],
                preferred_element_type=jnp.float32) + b2_ref[...]
    for k in range(no):
        y = jnp.dot(x_ref[k] * s[k:k + 1, :], w_ref[...],
                    preferred_element_type=jnp.float32)
        out_ref[k] = jnp.tanh(y)


def _genblock_call(x, H, W, style, A, b2, w_taps, wsq, ns, noise,
                   p_in=1, rg=1, no=None):
    Bg, T, cin_p = x.shape
    cin = cin_p // p_in
    cout = w_taps.shape[-1]
    nz = A.shape[0]
    hp, wp = H + 2, W + 2
    n_rows = hp * wp
    g = _guard(wp)
    if no is None:
        no = 16 if H <= 16 else (8 if H <= 32 else 2)
    grid = Bg // (no * rg)

    B = Bg * p_in
    nflat = jnp.pad(noise, ((0, 0), (1, 1), (1, 1), (0, 0)))
    nflat = nflat.reshape(grid, no * rg, p_in, n_rows)
    sty_p = style.reshape(grid, no * rg, p_in * nz)
    wt = jnp.stack([_kron_eye(w_taps[t], p_in) for t in range(9)], 0)
    wt = wt.astype(jnp.bfloat16)
    A_p = _kron_eye(A, p_in)
    b2_p = jnp.tile(b2, (1, p_in))
    wsq_p = _kron_eye(wsq, p_in)

    taps = tuple((ky - 1) * wp + (kx - 1) for ky in range(3) for kx in range(3))
    kern = functools.partial(_genblock_kernel, taps=taps, g=g, n_rows=n_rows,
                             hp=hp, wp=wp, p_in=p_in, rg=rg, no=no, cout=cout)
    return pl.pallas_call(
        kern,
        out_shape=jax.ShapeDtypeStruct((Bg // rg, T, rg * p_in * cout),
                                       jnp.float32),
        grid=(grid,),
        in_specs=[
            pl.BlockSpec((1, no * rg, p_in * nz), lambda i: (i, 0, 0)),
            pl.BlockSpec((no * rg, T, cin_p), lambda i: (i, 0, 0)),
            pl.BlockSpec((p_in * nz, cin_p), lambda i: (0, 0)),
            pl.BlockSpec((1, cin_p), lambda i: (0, 0)),
            pl.BlockSpec((9, cin_p, p_in * cout), lambda i: (0, 0, 0)),
            pl.BlockSpec((cin_p, p_in * cout), lambda i: (0, 0)),
            pl.BlockSpec((1, no * rg, p_in, n_rows), lambda i: (i, 0, 0, 0)),
            pl.BlockSpec(memory_space=pltpu.MemorySpace.SMEM),
        ],
        out_specs=pl.BlockSpec((no, T, rg * p_in * cout), lambda i: (i, 0, 0)),
        compiler_params=pltpu.CompilerParams(
            dimension_semantics=("parallel",),
            vmem_limit_bytes=_VMEM_LIMIT,
        ),
    )(sty_p, x, A_p, b2_p, wt, wsq_p, nflat, ns)


def _rgb_call(x, H, W, style, A, b2, w_mat, p_in):
    Bg, T, cin_p = x.shape
    nz = A.shape[0]
    hp, wp = H + 2, W + 2
    g = _guard(wp)
    no = 2
    B = Bg * p_in
    sty_p = style.reshape(Bg // no, no, p_in * nz)
    A_p = _kron_eye(A, p_in)
    b2_p = jnp.tile(b2, (1, p_in))
    w_p = _kron_eye(w_mat, p_in)
    out = pl.pallas_call(
        functools.partial(_rgb_kernel, no=no),
        out_shape=jax.ShapeDtypeStruct((Bg, T, p_in * 3), jnp.float32),
        grid=(Bg // no,),
        in_specs=[
            pl.BlockSpec((1, no, p_in * nz), lambda i: (i, 0, 0)),
            pl.BlockSpec((no, T, cin_p), lambda i: (i, 0, 0)),
            pl.BlockSpec((p_in * nz, cin_p), lambda i: (0, 0)),
            pl.BlockSpec((1, cin_p), lambda i: (0, 0)),
            pl.BlockSpec((cin_p, p_in * 3), lambda i: (0, 0)),
        ],
        out_specs=pl.BlockSpec((no, T, p_in * 3), lambda i: (i, 0, 0)),
        compiler_params=pltpu.CompilerParams(
            dimension_semantics=("parallel",),
            vmem_limit_bytes=_VMEM_LIMIT,
        ),
    )(sty_p, x, A_p, b2_p, w_p)
    o = out[:, g:g + hp * wp, :].reshape(Bg, hp, wp, p_in, 3)
    o = jnp.transpose(o, (0, 3, 1, 2, 4)).reshape(B, hp, wp, 3)
    return o[:, 1:1 + H, 1:1 + W, :]


def _upsample_padded(x, H, W):
    Bg, T, C = x.shape
    hp, wp = H + 2, W + 2
    g = _guard(wp)
    xi = x[:, g:g + hp * wp, :].reshape(Bg, hp, wp, C)[:, 1:1 + H, 1:1 + W, :]

    def shifted(v, axis, s):
        n = v.shape[axis]
        if s < 0:
            edge = lax.slice_in_dim(v, 0, 1, axis=axis)
            return jnp.concatenate([edge, lax.slice_in_dim(v, 0, n - 1, axis=axis)], axis)
        edge = lax.slice_in_dim(v, n - 1, n, axis=axis)
        return jnp.concatenate([lax.slice_in_dim(v, 1, n, axis=axis), edge], axis)

    up_ = shifted(xi, 1, -1)
    dn_ = shifted(xi, 1, 1)
    rows = (0.25 * up_ + 0.75 * xi, 0.75 * xi + 0.25 * dn_)
    phases = []
    for a in range(2):
        v = rows[a]
        lf = shifted(v, 2, -1)
        rt = shifted(v, 2, 1)
        phases.append(jnp.stack([0.25 * lf + 0.75 * v, 0.75 * v + 0.25 * rt], 3))
    xb = jnp.stack(phases, 2).reshape(Bg, 2 * H, 2 * W, C)
    H2, W2 = 2 * H, 2 * W
    g2 = _guard(W2 + 2)
    xb = jnp.pad(xb, ((0, 0), (1, 1), (1, 1), (0, 0)))
    xb = xb.reshape(Bg, (H2 + 2) * (W2 + 2), C)
    return jnp.pad(xb, ((0, 0), (g2, g2), (0, 0)))


def kernel(styles, const,
           b0_A, b0_b2, b0_w_taps, b0_wsq, b0_noise_strength,
           b1_A, b1_b2, b1_w_taps, b1_wsq, b1_noise_strength,
           b2_A, b2_b2, b2_w_taps, b2_wsq, b2_noise_strength,
           b3_A, b3_b2, b3_w_taps, b3_wsq, b3_noise_strength,
           b4_A, b4_b2, b4_w_taps, b4_wsq, b4_noise_strength,
           b5_A, b5_b2, b5_w_taps, b5_wsq, b5_noise_strength,
           b6_A, b6_b2, b6_w_taps, b6_wsq, b6_noise_strength,
           b7_A, b7_b2, b7_w_taps, b7_wsq, b7_noise_strength,
           b8_A, b8_b2, b8_w_taps, b8_wsq, b8_noise_strength,
           rgb_A, rgb_b2, rgb_w_mat,
           noise0, noise1, noise2, noise3, noise4,
           noise5, noise6, noise7, noise8):
    blocks = [
        (b0_A, b0_b2, b0_w_taps, b0_wsq, b0_noise_strength),
        (b1_A, b1_b2, b1_w_taps, b1_wsq, b1_noise_strength),
        (b2_A, b2_b2, b2_w_taps, b2_wsq, b2_noise_strength),
        (b3_A, b3_b2, b3_w_taps, b3_wsq, b3_noise_strength),
        (b4_A, b4_b2, b4_w_taps, b4_wsq, b4_noise_strength),
        (b5_A, b5_b2, b5_w_taps, b5_wsq, b5_noise_strength),
        (b6_A, b6_b2, b6_w_taps, b6_wsq, b6_noise_strength),
        (b7_A, b7_b2, b7_w_taps, b7_wsq, b7_noise_strength),
        (b8_A, b8_b2, b8_w_taps, b8_wsq, b8_noise_strength),
    ]
    noises = [noise0, noise1, noise2, noise3, noise4,
              noise5, noise6, noise7, noise8]
    plan = [
        (1, 1, 16), (1, 1, 16), (1, 1, 16), (1, 1, 16), (1, 1, 16),
        (1, 2, 4),
        (2, 2, 2),
        (4, 1, 1),
        (4, 1, 1),
    ]
    B = styles.shape[0]
    nf = const.shape[-1]

    H = W = 4
    g0 = _guard(W + 2)
    cflat = jnp.pad(const, ((1, 1), (1, 1), (0, 0))).reshape((H + 2) * (W + 2), nf)
    cflat = jnp.pad(cflat, ((g0, g0), (0, 0)))
    x = jnp.broadcast_to(cflat[None], (B, cflat.shape[0], nf))

    for j, (A, b2, w_taps, wsq, ns) in enumerate(blocks):
        if _UPSAMPLE[j]:
            x = _upsample_padded(x, H, W)
            H, W = 2 * H, 2 * W
        p_in, rg, no = plan[j]
        x = _genblock_call(x, H, W, styles[:, j, :], A, b2, w_taps, wsq, ns,
                           noises[j], p_in=p_in, rg=rg, no=no)
    return _rgb_call(x, H, W, styles[:, -1, :], rgb_A, rgb_b2, rgb_w_mat, p_in=4)


# speedup vs baseline: 2.7356x; 1.2759x over previous
import functools

import jax
import jax.numpy as jnp
from jax import lax
from jax.experimental import pallas as pl
from jax.experimental.pallas import tpu as pltpu

_EPS = 1e-8
_UPSAMPLE = (False, True, False, True, False, True, False, True, False)
_VMEM_LIMIT = int(min(96 * 1024 * 1024, (3 * (64 * 1024 * 1024)) // 4))


def _guard(wp):
    return ((wp + 2 + 7) // 8) * 8


def _kron_eye(w, p):
    if p == 1:
        return w
    a, b = w.shape
    return (jnp.eye(p, dtype=w.dtype)[:, None, :, None] *
            w[None, :, None, :]).reshape(p * a, p * b)


def _genblock_kernel(style_ref, x_ref, a_ref, b2_ref, wt_ref, wsq_ref, noise_ref,
                     ns_ref, out_ref, *, taps, g, n_rows, hp, wp, y0, x0, hh, ww,
                     p_in, rg, no, cout, up_out, g_out):
    cin_p = x_ref.shape[-1]
    cout_p = p_in * cout
    s = jnp.dot(style_ref[0], a_ref[...],
                preferred_element_type=jnp.float32) + b2_ref[...]
    d = lax.rsqrt(jnp.dot(s * s, wsq_ref[...],
                          preferred_element_type=jnp.float32) + _EPS)
    ns = ns_ref[0]
    r = lax.broadcasted_iota(jnp.int32, (n_rows, 1), 0)
    yc = r // wp
    xc = r - yc * wp
    interior = ((yc >= y0) & (yc < y0 + hh) & (xc >= x0) & (xc < x0 + ww))
    zg = jnp.zeros((g_out, rg * cout_p), jnp.float32)
    if p_in > 1:
        io_l = lax.broadcasted_iota(jnp.int32, (p_in, cout_p), 1)
        io_p = lax.broadcasted_iota(jnp.int32, (p_in, cout_p), 0)
        expand = (io_l // cout == io_p).astype(jnp.float32)
    for o in range(no):
        ys = []
        for q in range(rg):
            gi = o * rg + q
            xm = (x_ref[gi] * s[gi:gi + 1, :]).astype(jnp.bfloat16)
            acc = jnp.zeros((n_rows, cout_p), jnp.float32)
            for t, delta in enumerate(taps):
                start = g + delta
                acc = acc + jnp.dot(xm[start:start + n_rows, :], wt_ref[t],
                                    preferred_element_type=jnp.float32)
            if p_in > 1:
                nz = jnp.dot(jnp.transpose(noise_ref[0, gi]), expand,
                             preferred_element_type=jnp.float32)
            else:
                nz = jnp.transpose(noise_ref[0, gi])
            y = acc * d[gi:gi + 1, :] + ns * nz
            y = jnp.where(y >= 0.0, y, 0.2 * y)
            if not up_out:
                ys.append(jnp.where(interior, y, 0.0))
                continue
            sh_p1 = jnp.concatenate([y[1:], y[-1:]], 0)
            sh_m1 = jnp.concatenate([y[:1], y[:-1]], 0)
            vc = jnp.where(xc == x0 - 1, sh_p1,
                           jnp.where(xc == x0 + ww, sh_m1, y))
            sh_pw = jnp.concatenate([vc[wp:], vc[-wp:]], 0)
            sh_mw = jnp.concatenate([vc[:wp], vc[:-wp]], 0)
            vr = jnp.where(yc == y0 - 1, sh_pw,
                           jnp.where(yc == y0 + hh, sh_mw, vc))
            rowm = jnp.concatenate([vr[:wp], vr[:-wp]], 0)
            rowp = jnp.concatenate([vr[wp:], vr[-wp:]], 0)
            rows2 = (0.25 * rowm + 0.75 * vr, 0.75 * vr + 0.25 * rowp)
            xas = []
            for a in range(2):
                ra = rows2[a]
                colm = jnp.concatenate([ra[:1], ra[:-1]], 0)
                colp = jnp.concatenate([ra[1:], ra[-1:]], 0)
                pe = jnp.where(interior, 0.25 * colm + 0.75 * ra, 0.0)
                po = jnp.where(interior, 0.75 * ra + 0.25 * colp, 0.0)
                xa = jnp.stack([pe, po], 1).reshape(2 * n_rows, cout_p)
                xas.append(xa.reshape(hp, 2 * wp, cout_p))
            big = jnp.stack(xas, 1).reshape(4 * n_rows, cout_p)
            ys.append(big)
        y_all = ys[0] if rg == 1 else jnp.concatenate(ys, axis=1)
        out_ref[o] = jnp.concatenate([zg, y_all, zg], axis=0)


def _rgb_kernel(style_ref, x_ref, a_ref, b2_ref, w_ref, out_ref, *, no):
    s = jnp.dot(style_ref[0], a_ref[...],
                preferred_element_type=jnp.float32) + b2_ref[...]
    for k in range(no):
        y = jnp.dot(x_ref[k] * s[k:k + 1, :], w_ref[...],
                    preferred_element_type=jnp.float32)
        out_ref[k] = jnp.tanh(y)


def _genblock_call(x, geom, style, A, b2, w_taps, wsq, ns, noise,
                   p_in=1, rg=1, no=None, up_out=False):
    Bg, T, cin_p = x.shape
    hp, wp, y0, x0, H, W = geom
    cin = cin_p // p_in
    cout = w_taps.shape[-1]
    nz = A.shape[0]
    n_rows = hp * wp
    g = _guard(wp)
    if no is None:
        no = 16 if H <= 16 else (8 if H <= 32 else 2)
    grid = Bg // (no * rg)

    nflat = jnp.pad(noise, ((0, 0), (y0, hp - y0 - H), (x0, wp - x0 - W), (0, 0)))
    nflat = nflat.reshape(grid, no * rg, p_in, n_rows)
    sty_p = style.reshape(grid, no * rg, p_in * nz)
    wt = jnp.stack([_kron_eye(w_taps[t], p_in) for t in range(9)], 0)
    wt = wt.astype(jnp.bfloat16)
    A_p = _kron_eye(A, p_in)
    b2_p = jnp.tile(b2, (1, p_in))
    wsq_p = _kron_eye(wsq, p_in)

    if up_out:
        g_out = _guard(2 * wp)
        rows_out = 4 * n_rows
    else:
        g_out = g
        rows_out = n_rows
    T_out = 2 * g_out + rows_out

    taps = tuple((ky - 1) * wp + (kx - 1) for ky in range(3) for kx in range(3))
    kern = functools.partial(_genblock_kernel, taps=taps, g=g, n_rows=n_rows,
                             hp=hp, wp=wp, y0=y0, x0=x0, hh=H, ww=W,
                             p_in=p_in, rg=rg, no=no, cout=cout,
                             up_out=up_out, g_out=g_out)
    return pl.pallas_call(
        kern,
        out_shape=jax.ShapeDtypeStruct((Bg // rg, T_out, rg * p_in * cout),
                                       jnp.float32),
        grid=(grid,),
        in_specs=[
            pl.BlockSpec((1, no * rg, p_in * nz), lambda i: (i, 0, 0)),
            pl.BlockSpec((no * rg, T, cin_p), lambda i: (i, 0, 0)),
            pl.BlockSpec((p_in * nz, cin_p), lambda i: (0, 0)),
            pl.BlockSpec((1, cin_p), lambda i: (0, 0)),
            pl.BlockSpec((9, cin_p, p_in * cout), lambda i: (0, 0, 0)),
            pl.BlockSpec((cin_p, p_in * cout), lambda i: (0, 0)),
            pl.BlockSpec((1, no * rg, p_in, n_rows), lambda i: (i, 0, 0, 0)),
            pl.BlockSpec(memory_space=pltpu.MemorySpace.SMEM),
        ],
        out_specs=pl.BlockSpec((no, T_out, rg * p_in * cout), lambda i: (i, 0, 0)),
        compiler_params=pltpu.CompilerParams(
            dimension_semantics=("parallel",),
            vmem_limit_bytes=_VMEM_LIMIT,
        ),
    )(sty_p, x, A_p, b2_p, wt, wsq_p, nflat, ns)


def _rgb_call(x, geom, style, A, b2, w_mat, p_in):
    Bg, T, cin_p = x.shape
    hp, wp, y0, x0, H, W = geom
    nz = A.shape[0]
    g = _guard(wp)
    no = 2
    B = Bg * p_in
    sty_p = style.reshape(Bg // no, no, p_in * nz)
    A_p = _kron_eye(A, p_in)
    b2_p = jnp.tile(b2, (1, p_in))
    w_p = _kron_eye(w_mat, p_in)
    out = pl.pallas_call(
        functools.partial(_rgb_kernel, no=no),
        out_shape=jax.ShapeDtypeStruct((Bg, T, p_in * 3), jnp.float32),
        grid=(Bg // no,),
        in_specs=[
            pl.BlockSpec((1, no, p_in * nz), lambda i: (i, 0, 0)),
            pl.BlockSpec((no, T, cin_p), lambda i: (i, 0, 0)),
            pl.BlockSpec((p_in * nz, cin_p), lambda i: (0, 0)),
            pl.BlockSpec((1, cin_p), lambda i: (0, 0)),
            pl.BlockSpec((cin_p, p_in * 3), lambda i: (0, 0)),
        ],
        out_specs=pl.BlockSpec((no, T, p_in * 3), lambda i: (i, 0, 0)),
        compiler_params=pltpu.CompilerParams(
            dimension_semantics=("parallel",),
            vmem_limit_bytes=_VMEM_LIMIT,
        ),
    )(sty_p, x, A_p, b2_p, w_p)
    o = out[:, g:g + hp * wp, :].reshape(Bg, hp, wp, p_in, 3)
    o = jnp.transpose(o, (0, 3, 1, 2, 4)).reshape(B, hp, wp, 3)
    return o[:, y0:y0 + H, x0:x0 + W, :]


def _upsample_padded(x, H, W):
    Bg, T, C = x.shape
    hp, wp = H + 2, W + 2
    g = _guard(wp)
    xi = x[:, g:g + hp * wp, :].reshape(Bg, hp, wp, C)[:, 1:1 + H, 1:1 + W, :]

    def shifted(v, axis, s):
        n = v.shape[axis]
        if s < 0:
            edge = lax.slice_in_dim(v, 0, 1, axis=axis)
            return jnp.concatenate([edge, lax.slice_in_dim(v, 0, n - 1, axis=axis)], axis)
        edge = lax.slice_in_dim(v, n - 1, n, axis=axis)
        return jnp.concatenate([lax.slice_in_dim(v, 1, n, axis=axis), edge], axis)

    up_ = shifted(xi, 1, -1)
    dn_ = shifted(xi, 1, 1)
    rows = (0.25 * up_ + 0.75 * xi, 0.75 * xi + 0.25 * dn_)
    phases = []
    for a in range(2):
        v = rows[a]
        lf = shifted(v, 2, -1)
        rt = shifted(v, 2, 1)
        phases.append(jnp.stack([0.25 * lf + 0.75 * v, 0.75 * v + 0.25 * rt], 3))
    xb = jnp.stack(phases, 2).reshape(Bg, 2 * H, 2 * W, C)
    H2, W2 = 2 * H, 2 * W
    g2 = _guard(W2 + 2)
    xb = jnp.pad(xb, ((0, 0), (1, 1), (1, 1), (0, 0)))
    xb = xb.reshape(Bg, (H2 + 2) * (W2 + 2), C)
    return jnp.pad(xb, ((0, 0), (g2, g2), (0, 0)))


def kernel(styles, const,
           b0_A, b0_b2, b0_w_taps, b0_wsq, b0_noise_strength,
           b1_A, b1_b2, b1_w_taps, b1_wsq, b1_noise_strength,
           b2_A, b2_b2, b2_w_taps, b2_wsq, b2_noise_strength,
           b3_A, b3_b2, b3_w_taps, b3_wsq, b3_noise_strength,
           b4_A, b4_b2, b4_w_taps, b4_wsq, b4_noise_strength,
           b5_A, b5_b2, b5_w_taps, b5_wsq, b5_noise_strength,
           b6_A, b6_b2, b6_w_taps, b6_wsq, b6_noise_strength,
           b7_A, b7_b2, b7_w_taps, b7_wsq, b7_noise_strength,
           b8_A, b8_b2, b8_w_taps, b8_wsq, b8_noise_strength,
           rgb_A, rgb_b2, rgb_w_mat,
           noise0, noise1, noise2, noise3, noise4,
           noise5, noise6, noise7, noise8):
    blocks = [
        (b0_A, b0_b2, b0_w_taps, b0_wsq, b0_noise_strength),
        (b1_A, b1_b2, b1_w_taps, b1_wsq, b1_noise_strength),
        (b2_A, b2_b2, b2_w_taps, b2_wsq, b2_noise_strength),
        (b3_A, b3_b2, b3_w_taps, b3_wsq, b3_noise_strength),
        (b4_A, b4_b2, b4_w_taps, b4_wsq, b4_noise_strength),
        (b5_A, b5_b2, b5_w_taps, b5_wsq, b5_noise_strength),
        (b6_A, b6_b2, b6_w_taps, b6_wsq, b6_noise_strength),
        (b7_A, b7_b2, b7_w_taps, b7_wsq, b7_noise_strength),
        (b8_A, b8_b2, b8_w_taps, b8_wsq, b8_noise_strength),
    ]
    noises = [noise0, noise1, noise2, noise3, noise4,
              noise5, noise6, noise7, noise8]
    plan = [
        (1, 1, 16, False), (1, 1, 16, False), (1, 1, 16, False),
        (1, 1, 16, False), (1, 1, 16, False),
        (1, 2, 4, False),
        (2, 2, 1, True),
        (4, 1, 1, False),
        (4, 1, 1, False),
    ]
    B = styles.shape[0]
    nf = const.shape[-1]

    H = W = 4
    g0 = _guard(W + 2)
    cflat = jnp.pad(const, ((1, 1), (1, 1), (0, 0))).reshape((H + 2) * (W + 2), nf)
    cflat = jnp.pad(cflat, ((g0, g0), (0, 0)))
    x = jnp.broadcast_to(cflat[None], (B, cflat.shape[0], nf))
    geom = (H + 2, W + 2, 1, 1, H, W)

    pending_up = False
    for j, (A, b2, w_taps, wsq, ns) in enumerate(blocks):
        if _UPSAMPLE[j]:
            H, W = 2 * H, 2 * W
            if pending_up:
                hp, wp, y0, x0 = (2 * geom[0], 2 * geom[1],
                                  2 * geom[2], 2 * geom[3])
                geom = (hp, wp, y0, x0, H, W)
                pending_up = False
            else:
                x = _upsample_padded(x, H // 2, W // 2)
                geom = (H + 2, W + 2, 1, 1, H, W)
        p_in, rg, no, up_out = plan[j]
        x = _genblock_call(x, geom, styles[:, j, :], A, b2, w_taps, wsq, ns,
                           noises[j], p_in=p_in, rg=rg, no=no, up_out=up_out)
        if up_out:
            pending_up = True
    return _rgb_call(x, geom, styles[:, -1, :], rgb_A, rgb_b2, rgb_w_mat, p_in=4)


# speedup vs baseline: 2.9235x; 1.0687x over previous
import functools

import jax
import jax.numpy as jnp
from jax import lax
from jax.experimental import pallas as pl
from jax.experimental.pallas import tpu as pltpu

_EPS = 1e-8
_UPSAMPLE = (False, True, False, True, False, True, False, True, False)
_VMEM_LIMIT = int(min(96 * 1024 * 1024, (3 * (64 * 1024 * 1024)) // 4))


def _guard(wp):
    return ((wp + 2 + 7) // 8) * 8


def _kron_eye(w, p):
    if p == 1:
        return w
    a, b = w.shape
    return (jnp.eye(p, dtype=w.dtype)[:, None, :, None] *
            w[None, :, None, :]).reshape(p * a, p * b)


def _genblock_kernel(style_ref, x_ref, a_ref, b2_ref, wt_ref, wsq_ref, noise_ref,
                     ns_ref, out_ref, *, taps, g, n_rows, hp, wp, y0, x0, hh, ww,
                     p_in, rg, no, cout, up_out, trim_out, g_out):
    cin_p = x_ref.shape[-1]
    cout_p = p_in * cout
    s = jnp.dot(style_ref[0], a_ref[...],
                preferred_element_type=jnp.float32) + b2_ref[...]
    d = lax.rsqrt(jnp.dot(s * s, wsq_ref[...],
                          preferred_element_type=jnp.float32) + _EPS)
    ns = ns_ref[0]
    r = lax.broadcasted_iota(jnp.int32, (n_rows, 1), 0)
    yc = r // wp
    xc = r - yc * wp
    interior = ((yc >= y0) & (yc < y0 + hh) & (xc >= x0) & (xc < x0 + ww))
    zg = jnp.zeros((g_out, rg * cout_p), jnp.float32)
    if p_in > 1:
        io_l = lax.broadcasted_iota(jnp.int32, (p_in, cout_p), 1)
        io_p = lax.broadcasted_iota(jnp.int32, (p_in, cout_p), 0)
        expand = (io_l // cout == io_p).astype(jnp.float32)
    for o in range(no):
        ys = []
        for q in range(rg):
            gi = o * rg + q
            xm = (x_ref[gi] * s[gi:gi + 1, :]).astype(jnp.bfloat16)
            acc = jnp.zeros((n_rows, cout_p), jnp.float32)
            for t, delta in enumerate(taps):
                start = g + delta
                acc = acc + jnp.dot(xm[start:start + n_rows, :], wt_ref[t],
                                    preferred_element_type=jnp.float32)
            if p_in > 1:
                nz = jnp.dot(jnp.transpose(noise_ref[0, gi]), expand,
                             preferred_element_type=jnp.float32)
            else:
                nz = jnp.transpose(noise_ref[0, gi])
            y = acc * d[gi:gi + 1, :] + ns * nz
            y = jnp.where(y >= 0.0, y, 0.2 * y)
            if trim_out:
                ym = jnp.where(interior, y, 0.0).reshape(hp, wp, cout_p)
                ym = ym[y0 - 1:y0 + hh + 1, x0 - 1:x0 + ww + 1]
                ys.append(ym.reshape((hh + 2) * (ww + 2), cout_p))
                continue
            if not up_out:
                ys.append(jnp.where(interior, y, 0.0))
                continue
            sh_p1 = jnp.concatenate([y[1:], y[-1:]], 0)
            sh_m1 = jnp.concatenate([y[:1], y[:-1]], 0)
            vc = jnp.where(xc == x0 - 1, sh_p1,
                           jnp.where(xc == x0 + ww, sh_m1, y))
            sh_pw = jnp.concatenate([vc[wp:], vc[-wp:]], 0)
            sh_mw = jnp.concatenate([vc[:wp], vc[:-wp]], 0)
            vr = jnp.where(yc == y0 - 1, sh_pw,
                           jnp.where(yc == y0 + hh, sh_mw, vc))
            rowm = jnp.concatenate([vr[:wp], vr[:-wp]], 0)
            rowp = jnp.concatenate([vr[wp:], vr[-wp:]], 0)
            rows2 = (0.25 * rowm + 0.75 * vr, 0.75 * vr + 0.25 * rowp)
            xas = []
            for a in range(2):
                ra = rows2[a]
                colm = jnp.concatenate([ra[:1], ra[:-1]], 0)
                colp = jnp.concatenate([ra[1:], ra[-1:]], 0)
                pe = jnp.where(interior, 0.25 * colm + 0.75 * ra, 0.0)
                po = jnp.where(interior, 0.75 * ra + 0.25 * colp, 0.0)
                xa = jnp.stack([pe, po], 1).reshape(2 * n_rows, cout_p)
                xas.append(xa.reshape(hp, 2 * wp, cout_p))
            big = jnp.stack(xas, 1).reshape(4 * n_rows, cout_p)
            ys.append(big)
        y_all = ys[0] if rg == 1 else jnp.concatenate(ys, axis=1)
        out_ref[o] = jnp.concatenate([zg, y_all, zg], axis=0)


def _rgb_kernel(style_ref, x_ref, a_ref, b2_ref, w_ref, out_ref, *, no):
    s = jnp.dot(style_ref[0], a_ref[...],
                preferred_element_type=jnp.float32) + b2_ref[...]
    for k in range(no):
        y = jnp.dot(x_ref[k] * s[k:k + 1, :], w_ref[...],
                    preferred_element_type=jnp.float32)
        out_ref[k] = jnp.tanh(y)


def _genblock_call(x, geom, style, A, b2, w_taps, wsq, ns, noise,
                   p_in=1, rg=1, no=None, up_out=False, trim_out=False):
    Bg, T, cin_p = x.shape
    hp, wp, y0, x0, H, W = geom
    cin = cin_p // p_in
    cout = w_taps.shape[-1]
    nz = A.shape[0]
    n_rows = hp * wp
    g = _guard(wp)
    if no is None:
        no = 16 if H <= 16 else (8 if H <= 32 else 2)
    grid = Bg // (no * rg)

    nflat = jnp.pad(noise, ((0, 0), (y0, hp - y0 - H), (x0, wp - x0 - W), (0, 0)))
    nflat = nflat.reshape(grid, no * rg, p_in, n_rows)
    sty_p = style.reshape(grid, no * rg, p_in * nz)
    wt = jnp.stack([_kron_eye(w_taps[t], p_in) for t in range(9)], 0)
    wt = wt.astype(jnp.bfloat16)
    A_p = _kron_eye(A, p_in)
    b2_p = jnp.tile(b2, (1, p_in))
    wsq_p = _kron_eye(wsq, p_in)

    if up_out:
        g_out = _guard(2 * wp)
        rows_out = 4 * n_rows
    elif trim_out:
        g_out = _guard(W + 2)
        rows_out = (H + 2) * (W + 2)
    else:
        g_out = g
        rows_out = n_rows
    T_out = 2 * g_out + rows_out

    taps = tuple((ky - 1) * wp + (kx - 1) for ky in range(3) for kx in range(3))
    kern = functools.partial(_genblock_kernel, taps=taps, g=g, n_rows=n_rows,
                             hp=hp, wp=wp, y0=y0, x0=x0, hh=H, ww=W,
                             p_in=p_in, rg=rg, no=no, cout=cout,
                             up_out=up_out, trim_out=trim_out, g_out=g_out)
    return pl.pallas_call(
        kern,
        out_shape=jax.ShapeDtypeStruct((Bg // rg, T_out, rg * p_in * cout),
                                       jnp.float32),
        grid=(grid,),
        in_specs=[
            pl.BlockSpec((1, no * rg, p_in * nz), lambda i: (i, 0, 0)),
            pl.BlockSpec((no * rg, T, cin_p), lambda i: (i, 0, 0)),
            pl.BlockSpec((p_in * nz, cin_p), lambda i: (0, 0)),
            pl.BlockSpec((1, cin_p), lambda i: (0, 0)),
            pl.BlockSpec((9, cin_p, p_in * cout), lambda i: (0, 0, 0)),
            pl.BlockSpec((cin_p, p_in * cout), lambda i: (0, 0)),
            pl.BlockSpec((1, no * rg, p_in, n_rows), lambda i: (i, 0, 0, 0)),
            pl.BlockSpec(memory_space=pltpu.MemorySpace.SMEM),
        ],
        out_specs=pl.BlockSpec((no, T_out, rg * p_in * cout), lambda i: (i, 0, 0)),
        compiler_params=pltpu.CompilerParams(
            dimension_semantics=("parallel",),
            vmem_limit_bytes=_VMEM_LIMIT,
        ),
    )(sty_p, x, A_p, b2_p, wt, wsq_p, nflat, ns)


def _rgb_call(x, geom, style, A, b2, w_mat, p_in):
    Bg, T, cin_p = x.shape
    hp, wp, y0, x0, H, W = geom
    nz = A.shape[0]
    g = _guard(wp)
    no = 2
    B = Bg * p_in
    sty_p = style.reshape(Bg // no, no, p_in * nz)
    A_p = _kron_eye(A, p_in)
    b2_p = jnp.tile(b2, (1, p_in))
    w_p = _kron_eye(w_mat, p_in)
    out = pl.pallas_call(
        functools.partial(_rgb_kernel, no=no),
        out_shape=jax.ShapeDtypeStruct((Bg, T, p_in * 3), jnp.float32),
        grid=(Bg // no,),
        in_specs=[
            pl.BlockSpec((1, no, p_in * nz), lambda i: (i, 0, 0)),
            pl.BlockSpec((no, T, cin_p), lambda i: (i, 0, 0)),
            pl.BlockSpec((p_in * nz, cin_p), lambda i: (0, 0)),
            pl.BlockSpec((1, cin_p), lambda i: (0, 0)),
            pl.BlockSpec((cin_p, p_in * 3), lambda i: (0, 0)),
        ],
        out_specs=pl.BlockSpec((no, T, p_in * 3), lambda i: (i, 0, 0)),
        compiler_params=pltpu.CompilerParams(
            dimension_semantics=("parallel",),
            vmem_limit_bytes=_VMEM_LIMIT,
        ),
    )(sty_p, x, A_p, b2_p, w_p)
    o = out[:, g:g + hp * wp, :].reshape(Bg, hp, wp, p_in, 3)
    o = jnp.transpose(o, (0, 3, 1, 2, 4)).reshape(B, hp, wp, 3)
    return o[:, y0:y0 + H, x0:x0 + W, :]


def kernel(styles, const,
           b0_A, b0_b2, b0_w_taps, b0_wsq, b0_noise_strength,
           b1_A, b1_b2, b1_w_taps, b1_wsq, b1_noise_strength,
           b2_A, b2_b2, b2_w_taps, b2_wsq, b2_noise_strength,
           b3_A, b3_b2, b3_w_taps, b3_wsq, b3_noise_strength,
           b4_A, b4_b2, b4_w_taps, b4_wsq, b4_noise_strength,
           b5_A, b5_b2, b5_w_taps, b5_wsq, b5_noise_strength,
           b6_A, b6_b2, b6_w_taps, b6_wsq, b6_noise_strength,
           b7_A, b7_b2, b7_w_taps, b7_wsq, b7_noise_strength,
           b8_A, b8_b2, b8_w_taps, b8_wsq, b8_noise_strength,
           rgb_A, rgb_b2, rgb_w_mat,
           noise0, noise1, noise2, noise3, noise4,
           noise5, noise6, noise7, noise8):
    blocks = [
        (b0_A, b0_b2, b0_w_taps, b0_wsq, b0_noise_strength),
        (b1_A, b1_b2, b1_w_taps, b1_wsq, b1_noise_strength),
        (b2_A, b2_b2, b2_w_taps, b2_wsq, b2_noise_strength),
        (b3_A, b3_b2, b3_w_taps, b3_wsq, b3_noise_strength),
        (b4_A, b4_b2, b4_w_taps, b4_wsq, b4_noise_strength),
        (b5_A, b5_b2, b5_w_taps, b5_wsq, b5_noise_strength),
        (b6_A, b6_b2, b6_w_taps, b6_wsq, b6_noise_strength),
        (b7_A, b7_b2, b7_w_taps, b7_wsq, b7_noise_strength),
        (b8_A, b8_b2, b8_w_taps, b8_wsq, b8_noise_strength),
    ]
    noises = [noise0, noise1, noise2, noise3, noise4,
              noise5, noise6, noise7, noise8]
    plan = [
        (1, 1, 16, True, False),
        (1, 1, 16, False, True),
        (1, 1, 16, True, False),
        (1, 1, 16, False, True),
        (1, 1, 16, True, False),
        (1, 2, 4, False, True),
        (2, 2, 1, True, False),
        (4, 1, 1, False, True),
        (4, 1, 1, False, False),
    ]
    B = styles.shape[0]
    nf = const.shape[-1]

    H = W = 4
    g0 = _guard(W + 2)
    cflat = jnp.pad(const, ((1, 1), (1, 1), (0, 0))).reshape((H + 2) * (W + 2), nf)
    cflat = jnp.pad(cflat, ((g0, g0), (0, 0)))
    x = jnp.broadcast_to(cflat[None], (B, cflat.shape[0], nf))
    geom = (H + 2, W + 2, 1, 1, H, W)

    for j, (A, b2, w_taps, wsq, ns) in enumerate(blocks):
        if _UPSAMPLE[j]:
            H, W = 2 * H, 2 * W
            geom = (2 * geom[0], 2 * geom[1], 2 * geom[2], 2 * geom[3], H, W)
        p_in, rg, no, up_out, trim_out = plan[j]
        x = _genblock_call(x, geom, styles[:, j, :], A, b2, w_taps, wsq, ns,
                           noises[j], p_in=p_in, rg=rg, no=no, up_out=up_out,
                           trim_out=trim_out)
        if trim_out:
            geom = (H + 2, W + 2, 1, 1, H, W)
    return _rgb_call(x, geom, styles[:, -1, :], rgb_A, rgb_b2, rgb_w_mat, p_in=4)


# speedup vs baseline: 3.1696x; 1.0842x over previous
import functools

import jax
import jax.numpy as jnp
from jax import lax
from jax.experimental import pallas as pl
from jax.experimental.pallas import tpu as pltpu

_EPS = 1e-8
_UPSAMPLE = (False, True, False, True, False, True, False, True, False)
_VMEM_LIMIT = int(min(96 * 1024 * 1024, (3 * (64 * 1024 * 1024)) // 4))


def _guard(wp):
    return ((wp + 2 + 7) // 8) * 8


def _kron_eye(w, p):
    if p == 1:
        return w
    a, b = w.shape
    return (jnp.eye(p, dtype=w.dtype)[:, None, :, None] *
            w[None, :, None, :]).reshape(p * a, p * b)


def _genblock_kernel(style_ref, x_ref, a_ref, b2_ref, wt_ref, wsq_ref, noise_ref,
                     ns_ref, out_ref, *, taps, g, n_rows, hp, wp, y0, x0, hh, ww,
                     p_in, rg, no, cout, up_out, trim_out, g_out):
    cin_p = x_ref.shape[-1]
    cout_p = p_in * cout
    s = jnp.dot(style_ref[0], a_ref[...],
                preferred_element_type=jnp.float32) + b2_ref[...]
    d = lax.rsqrt(jnp.dot(s * s, wsq_ref[...],
                          preferred_element_type=jnp.float32) + _EPS)
    ns = ns_ref[0]
    r = lax.broadcasted_iota(jnp.int32, (n_rows, 1), 0)
    yc = r // wp
    xc = r - yc * wp
    interior = ((yc >= y0) & (yc < y0 + hh) & (xc >= x0) & (xc < x0 + ww))
    zg = jnp.zeros((g_out, rg * cout_p), jnp.float32)
    if p_in > 1:
        io_l = lax.broadcasted_iota(jnp.int32, (p_in, cout_p), 1)
        io_p = lax.broadcasted_iota(jnp.int32, (p_in, cout_p), 0)
        expand = (io_l // cout == io_p).astype(jnp.float32)
    c_all = rg * cout_p
    for o in range(no):
        ys = []
        for q in range(rg):
            gi = o * rg + q
            xm = (x_ref[gi] * s[gi:gi + 1, :]).astype(jnp.bfloat16)
            acc = jnp.zeros((n_rows, cout_p), jnp.float32)
            for t, delta in enumerate(taps):
                start = g + delta
                acc = acc + jnp.dot(xm[start:start + n_rows, :], wt_ref[t],
                                    preferred_element_type=jnp.float32)
            if p_in > 1:
                nz = jnp.dot(jnp.transpose(noise_ref[0, gi]), expand,
                             preferred_element_type=jnp.float32)
            else:
                nz = jnp.transpose(noise_ref[0, gi])
            y = acc * d[gi:gi + 1, :] + ns * nz
            ys.append(jnp.where(y >= 0.0, y, 0.2 * y))
        y = ys[0] if rg == 1 else jnp.concatenate(ys, axis=1)
        if trim_out:
            ym = jnp.where(interior, y, 0.0).reshape(hp, wp, c_all)
            ym = ym[y0 - 1:y0 + hh + 1, x0 - 1:x0 + ww + 1]
            y_all = ym.reshape((hh + 2) * (ww + 2), c_all)
        elif not up_out:
            y_all = jnp.where(interior, y, 0.0)
        else:
            sh_p1 = jnp.concatenate([y[1:], y[-1:]], 0)
            sh_m1 = jnp.concatenate([y[:1], y[:-1]], 0)
            vc = jnp.where(xc == x0 - 1, sh_p1,
                           jnp.where(xc == x0 + ww, sh_m1, y))
            sh_pw = jnp.concatenate([vc[wp:], vc[-wp:]], 0)
            sh_mw = jnp.concatenate([vc[:wp], vc[:-wp]], 0)
            vr = jnp.where(yc == y0 - 1, sh_pw,
                           jnp.where(yc == y0 + hh, sh_mw, vc))
            rowm = jnp.concatenate([vr[:wp], vr[:-wp]], 0)
            rowp = jnp.concatenate([vr[wp:], vr[-wp:]], 0)
            rows2 = (0.25 * rowm + 0.75 * vr, 0.75 * vr + 0.25 * rowp)
            xas = []
            for a in range(2):
                ra = rows2[a]
                colm = jnp.concatenate([ra[:1], ra[:-1]], 0)
                colp = jnp.concatenate([ra[1:], ra[-1:]], 0)
                pe = jnp.where(interior, 0.25 * colm + 0.75 * ra, 0.0)
                po = jnp.where(interior, 0.75 * ra + 0.25 * colp, 0.0)
                xa = jnp.stack([pe, po], 1).reshape(2 * n_rows, c_all)
                xas.append(xa.reshape(hp, 2 * wp, c_all))
            y_all = jnp.stack(xas, 1).reshape(4 * n_rows, c_all)
        out_ref[o] = jnp.concatenate([zg, y_all, zg], axis=0)


def _rgb_kernel(style_ref, x_ref, a_ref, b2_ref, w_ref, out_ref, *, no):
    s = jnp.dot(style_ref[0], a_ref[...],
                preferred_element_type=jnp.float32) + b2_ref[...]
    for k in range(no):
        y = jnp.dot(x_ref[k] * s[k:k + 1, :], w_ref[...],
                    preferred_element_type=jnp.float32)
        out_ref[k] = jnp.tanh(y)


def _genblock_call(x, geom, style, A, b2, w_taps, wsq, ns, noise,
                   p_in=1, rg=1, no=None, up_out=False, trim_out=False):
    Bg, T, cin_p = x.shape
    hp, wp, y0, x0, H, W = geom
    cin = cin_p // p_in
    cout = w_taps.shape[-1]
    nz = A.shape[0]
    n_rows = hp * wp
    g = _guard(wp)
    if no is None:
        no = 16 if H <= 16 else (8 if H <= 32 else 2)
    grid = Bg // (no * rg)

    nflat = jnp.pad(noise, ((0, 0), (y0, hp - y0 - H), (x0, wp - x0 - W), (0, 0)))
    nflat = nflat.reshape(grid, no * rg, p_in, n_rows)
    sty_p = style.reshape(grid, no * rg, p_in * nz)
    wt = jnp.stack([_kron_eye(w_taps[t], p_in) for t in range(9)], 0)
    wt = wt.astype(jnp.bfloat16)
    A_p = _kron_eye(A, p_in)
    b2_p = jnp.tile(b2, (1, p_in))
    wsq_p = _kron_eye(wsq, p_in)

    if up_out:
        g_out = _guard(2 * wp)
        rows_out = 4 * n_rows
    elif trim_out:
        g_out = _guard(W + 2)
        rows_out = (H + 2) * (W + 2)
    else:
        g_out = g
        rows_out = n_rows
    T_out = 2 * g_out + rows_out

    taps = tuple((ky - 1) * wp + (kx - 1) for ky in range(3) for kx in range(3))
    kern = functools.partial(_genblock_kernel, taps=taps, g=g, n_rows=n_rows,
                             hp=hp, wp=wp, y0=y0, x0=x0, hh=H, ww=W,
                             p_in=p_in, rg=rg, no=no, cout=cout,
                             up_out=up_out, trim_out=trim_out, g_out=g_out)
    return pl.pallas_call(
        kern,
        out_shape=jax.ShapeDtypeStruct((Bg // rg, T_out, rg * p_in * cout),
                                       jnp.float32),
        grid=(grid,),
        in_specs=[
            pl.BlockSpec((1, no * rg, p_in * nz), lambda i: (i, 0, 0)),
            pl.BlockSpec((no * rg, T, cin_p), lambda i: (i, 0, 0)),
            pl.BlockSpec((p_in * nz, cin_p), lambda i: (0, 0)),
            pl.BlockSpec((1, cin_p), lambda i: (0, 0)),
            pl.BlockSpec((9, cin_p, p_in * cout), lambda i: (0, 0, 0)),
            pl.BlockSpec((cin_p, p_in * cout), lambda i: (0, 0)),
            pl.BlockSpec((1, no * rg, p_in, n_rows), lambda i: (i, 0, 0, 0)),
            pl.BlockSpec(memory_space=pltpu.MemorySpace.SMEM),
        ],
        out_specs=pl.BlockSpec((no, T_out, rg * p_in * cout), lambda i: (i, 0, 0)),
        compiler_params=pltpu.CompilerParams(
            dimension_semantics=("parallel",),
            vmem_limit_bytes=_VMEM_LIMIT,
        ),
    )(sty_p, x, A_p, b2_p, wt, wsq_p, nflat, ns)


def _rgb_call(x, geom, style, A, b2, w_mat, p_in):
    Bg, T, cin_p = x.shape
    hp, wp, y0, x0, H, W = geom
    nz = A.shape[0]
    g = _guard(wp)
    no = 2
    B = Bg * p_in
    sty_p = style.reshape(Bg // no, no, p_in * nz)
    A_p = _kron_eye(A, p_in)
    b2_p = jnp.tile(b2, (1, p_in))
    w_p = _kron_eye(w_mat, p_in)
    out = pl.pallas_call(
        functools.partial(_rgb_kernel, no=no),
        out_shape=jax.ShapeDtypeStruct((Bg, T, p_in * 3), jnp.float32),
        grid=(Bg // no,),
        in_specs=[
            pl.BlockSpec((1, no, p_in * nz), lambda i: (i, 0, 0)),
            pl.BlockSpec((no, T, cin_p), lambda i: (i, 0, 0)),
            pl.BlockSpec((p_in * nz, cin_p), lambda i: (0, 0)),
            pl.BlockSpec((1, cin_p), lambda i: (0, 0)),
            pl.BlockSpec((cin_p, p_in * 3), lambda i: (0, 0)),
        ],
        out_specs=pl.BlockSpec((no, T, p_in * 3), lambda i: (i, 0, 0)),
        compiler_params=pltpu.CompilerParams(
            dimension_semantics=("parallel",),
            vmem_limit_bytes=_VMEM_LIMIT,
        ),
    )(sty_p, x, A_p, b2_p, w_p)
    o = out[:, g:g + hp * wp, :].reshape(Bg, hp, wp, p_in, 3)
    o = jnp.transpose(o, (0, 3, 1, 2, 4)).reshape(B, hp, wp, 3)
    return o[:, y0:y0 + H, x0:x0 + W, :]


def kernel(styles, const,
           b0_A, b0_b2, b0_w_taps, b0_wsq, b0_noise_strength,
           b1_A, b1_b2, b1_w_taps, b1_wsq, b1_noise_strength,
           b2_A, b2_b2, b2_w_taps, b2_wsq, b2_noise_strength,
           b3_A, b3_b2, b3_w_taps, b3_wsq, b3_noise_strength,
           b4_A, b4_b2, b4_w_taps, b4_wsq, b4_noise_strength,
           b5_A, b5_b2, b5_w_taps, b5_wsq, b5_noise_strength,
           b6_A, b6_b2, b6_w_taps, b6_wsq, b6_noise_strength,
           b7_A, b7_b2, b7_w_taps, b7_wsq, b7_noise_strength,
           b8_A, b8_b2, b8_w_taps, b8_wsq, b8_noise_strength,
           rgb_A, rgb_b2, rgb_w_mat,
           noise0, noise1, noise2, noise3, noise4,
           noise5, noise6, noise7, noise8):
    blocks = [
        (b0_A, b0_b2, b0_w_taps, b0_wsq, b0_noise_strength),
        (b1_A, b1_b2, b1_w_taps, b1_wsq, b1_noise_strength),
        (b2_A, b2_b2, b2_w_taps, b2_wsq, b2_noise_strength),
        (b3_A, b3_b2, b3_w_taps, b3_wsq, b3_noise_strength),
        (b4_A, b4_b2, b4_w_taps, b4_wsq, b4_noise_strength),
        (b5_A, b5_b2, b5_w_taps, b5_wsq, b5_noise_strength),
        (b6_A, b6_b2, b6_w_taps, b6_wsq, b6_noise_strength),
        (b7_A, b7_b2, b7_w_taps, b7_wsq, b7_noise_strength),
        (b8_A, b8_b2, b8_w_taps, b8_wsq, b8_noise_strength),
    ]
    noises = [noise0, noise1, noise2, noise3, noise4,
              noise5, noise6, noise7, noise8]
    plan = [
        (1, 1, 16, True, False),
        (1, 1, 16, False, True),
        (1, 1, 16, True, False),
        (1, 1, 16, False, True),
        (1, 1, 16, True, False),
        (1, 2, 4, False, True),
        (2, 2, 1, True, False),
        (4, 1, 1, False, False),
        (4, 1, 1, False, False),
    ]
    B = styles.shape[0]
    nf = const.shape[-1]

    H = W = 4
    g0 = _guard(W + 2)
    cflat = jnp.pad(const, ((1, 1), (1, 1), (0, 0))).reshape((H + 2) * (W + 2), nf)
    cflat = jnp.pad(cflat, ((g0, g0), (0, 0)))
    x = jnp.broadcast_to(cflat[None], (B, cflat.shape[0], nf))
    geom = (H + 2, W + 2, 1, 1, H, W)

    for j, (A, b2, w_taps, wsq, ns) in enumerate(blocks):
        if _UPSAMPLE[j]:
            H, W = 2 * H, 2 * W
            geom = (2 * geom[0], 2 * geom[1], 2 * geom[2], 2 * geom[3], H, W)
        p_in, rg, no, up_out, trim_out = plan[j]
        x = _genblock_call(x, geom, styles[:, j, :], A, b2, w_taps, wsq, ns,
                           noises[j], p_in=p_in, rg=rg, no=no, up_out=up_out,
                           trim_out=trim_out)
        if trim_out:
            geom = (H + 2, W + 2, 1, 1, H, W)
    return _rgb_call(x, geom, styles[:, -1, :], rgb_A, rgb_b2, rgb_w_mat, p_in=4)


# speedup vs baseline: 3.2564x; 1.0274x over previous
import functools

import jax
import jax.numpy as jnp
from jax import lax
from jax.experimental import pallas as pl
from jax.experimental.pallas import tpu as pltpu

_EPS = 1e-8
_UPSAMPLE = (False, True, False, True, False, True, False, True, False)
_VMEM_LIMIT = int(min(96 * 1024 * 1024, (3 * (64 * 1024 * 1024)) // 4))


def _guard(wp):
    return ((wp + 2 + 7) // 8) * 8


def _kron_eye(w, p):
    if p == 1:
        return w
    a, b = w.shape
    return (jnp.eye(p, dtype=w.dtype)[:, None, :, None] *
            w[None, :, None, :]).reshape(p * a, p * b)


def _genblock_kernel(style_ref, x_ref, a_ref, b2_ref, wt_ref, wsq_ref, noise_ref,
                     ns_ref, out_ref, *, taps, g, n_rows, hp, wp, y0, x0, hh, ww,
                     p_in, rg, no, cout, up_out, trim_out, g_out):
    cin_p = x_ref.shape[-1]
    cout_p = p_in * cout
    s = jnp.dot(style_ref[0], a_ref[...],
                preferred_element_type=jnp.float32) + b2_ref[...]
    d = lax.rsqrt(jnp.dot(s * s, wsq_ref[...],
                          preferred_element_type=jnp.float32) + _EPS)
    ns = ns_ref[0]
    r = lax.broadcasted_iota(jnp.int32, (n_rows, 1), 0)
    yc = r // wp
    xc = r - yc * wp
    interior = ((yc >= y0) & (yc < y0 + hh) & (xc >= x0) & (xc < x0 + ww))
    zg = jnp.zeros((g_out, rg * cout_p), jnp.float32)
    if p_in > 1:
        io_l = lax.broadcasted_iota(jnp.int32, (p_in, cout_p), 1)
        io_p = lax.broadcasted_iota(jnp.int32, (p_in, cout_p), 0)
        expand = (io_l // cout == io_p).astype(jnp.float32)
    c_all = rg * cout_p
    for o in range(no):
        ys = []
        for q in range(rg):
            gi = o * rg + q
            xm = (x_ref[gi] * s[gi:gi + 1, :]).astype(jnp.bfloat16)
            acc = jnp.zeros((n_rows, cout_p), jnp.float32)
            for t, delta in enumerate(taps):
                start = g + delta
                acc = acc + jnp.dot(xm[start:start + n_rows, :], wt_ref[t],
                                    preferred_element_type=jnp.float32)
            if p_in > 1:
                nz = jnp.dot(jnp.transpose(noise_ref[0, gi]), expand,
                             preferred_element_type=jnp.float32)
            else:
                nz = jnp.transpose(noise_ref[0, gi])
            y = acc * d[gi:gi + 1, :] + ns * nz
            ys.append(jnp.where(y >= 0.0, y, 0.2 * y))
        y = ys[0] if rg == 1 else jnp.concatenate(ys, axis=1)
        if trim_out:
            ym = jnp.where(interior, y, 0.0).reshape(hp, wp, c_all)
            ym = ym[y0 - 1:y0 + hh + 1, x0 - 1:x0 + ww + 1]
            y_all = ym.reshape((hh + 2) * (ww + 2), c_all)
        elif not up_out:
            y_all = jnp.where(interior, y, 0.0)
        else:
            sh_p1 = jnp.concatenate([y[1:], y[-1:]], 0)
            sh_m1 = jnp.concatenate([y[:1], y[:-1]], 0)
            vc = jnp.where(xc == x0 - 1, sh_p1,
                           jnp.where(xc == x0 + ww, sh_m1, y))
            sh_pw = jnp.concatenate([vc[wp:], vc[-wp:]], 0)
            sh_mw = jnp.concatenate([vc[:wp], vc[:-wp]], 0)
            vr = jnp.where(yc == y0 - 1, sh_pw,
                           jnp.where(yc == y0 + hh, sh_mw, vc))
            rowm = jnp.concatenate([vr[:wp], vr[:-wp]], 0)
            rowp = jnp.concatenate([vr[wp:], vr[-wp:]], 0)
            rows2 = (0.25 * rowm + 0.75 * vr, 0.75 * vr + 0.25 * rowp)
            xas = []
            for a in range(2):
                ra = rows2[a]
                colm = jnp.concatenate([ra[:1], ra[:-1]], 0)
                colp = jnp.concatenate([ra[1:], ra[-1:]], 0)
                pe = jnp.where(interior, 0.25 * colm + 0.75 * ra, 0.0)
                po = jnp.where(interior, 0.75 * ra + 0.25 * colp, 0.0)
                xa = jnp.stack([pe, po], 1).reshape(2 * n_rows, c_all)
                xas.append(xa.reshape(hp, 2 * wp, c_all))
            y_all = jnp.stack(xas, 1).reshape(4 * n_rows, c_all)
        out_ref[o] = jnp.concatenate([zg, y_all, zg], axis=0)


def _rgb_kernel(style_ref, x_ref, a_ref, b2_ref, w_ref, out_ref, *, no):
    s = jnp.dot(style_ref[0], a_ref[...],
                preferred_element_type=jnp.float32) + b2_ref[...]
    for k in range(no):
        y = jnp.dot(x_ref[k] * s[k:k + 1, :], w_ref[...],
                    preferred_element_type=jnp.float32)
        out_ref[k] = jnp.tanh(y)


def _genblock_call(x, geom, style, A, b2, w_taps, wsq, ns, noise,
                   p_in=1, rg=1, no=None, up_out=False, trim_out=False):
    Bg, T, cin_p = x.shape
    hp, wp, y0, x0, H, W = geom
    cin = cin_p // p_in
    cout = w_taps.shape[-1]
    nz = A.shape[0]
    n_rows = hp * wp
    g = _guard(wp)
    if no is None:
        no = 16 if H <= 16 else (8 if H <= 32 else 2)
    grid = Bg // (no * rg)

    nflat = jnp.pad(noise, ((0, 0), (y0, hp - y0 - H), (x0, wp - x0 - W), (0, 0)))
    nflat = nflat.reshape(grid, no * rg, p_in, n_rows)
    sty_p = style.reshape(grid, no * rg, p_in * nz)
    wt = jnp.stack([_kron_eye(w_taps[t], p_in) for t in range(9)], 0)
    wt = wt.astype(jnp.bfloat16)
    A_p = _kron_eye(A, p_in)
    b2_p = jnp.tile(b2, (1, p_in))
    wsq_p = _kron_eye(wsq, p_in)

    if up_out:
        g_out = _guard(2 * wp)
        rows_out = 4 * n_rows
    elif trim_out:
        g_out = _guard(W + 2)
        rows_out = (H + 2) * (W + 2)
    else:
        g_out = g
        rows_out = n_rows
    T_out = 2 * g_out + rows_out

    taps = tuple((ky - 1) * wp + (kx - 1) for ky in range(3) for kx in range(3))
    kern = functools.partial(_genblock_kernel, taps=taps, g=g, n_rows=n_rows,
                             hp=hp, wp=wp, y0=y0, x0=x0, hh=H, ww=W,
                             p_in=p_in, rg=rg, no=no, cout=cout,
                             up_out=up_out, trim_out=trim_out, g_out=g_out)
    return pl.pallas_call(
        kern,
        out_shape=jax.ShapeDtypeStruct((Bg // rg, T_out, rg * p_in * cout),
                                       jnp.float32),
        grid=(grid,),
        in_specs=[
            pl.BlockSpec((1, no * rg, p_in * nz), lambda i: (i, 0, 0)),
            pl.BlockSpec((no * rg, T, cin_p), lambda i: (i, 0, 0)),
            pl.BlockSpec((p_in * nz, cin_p), lambda i: (0, 0)),
            pl.BlockSpec((1, cin_p), lambda i: (0, 0)),
            pl.BlockSpec((9, cin_p, p_in * cout), lambda i: (0, 0, 0)),
            pl.BlockSpec((cin_p, p_in * cout), lambda i: (0, 0)),
            pl.BlockSpec((1, no * rg, p_in, n_rows), lambda i: (i, 0, 0, 0)),
            pl.BlockSpec(memory_space=pltpu.MemorySpace.SMEM),
        ],
        out_specs=pl.BlockSpec((no, T_out, rg * p_in * cout), lambda i: (i, 0, 0)),
        compiler_params=pltpu.CompilerParams(
            dimension_semantics=("parallel",),
            vmem_limit_bytes=_VMEM_LIMIT,
        ),
    )(sty_p, x, A_p, b2_p, wt, wsq_p, nflat, ns)


def _rgb_call(x, geom, style, A, b2, w_mat, p_in):
    Bg, T, cin_p = x.shape
    hp, wp, y0, x0, H, W = geom
    nz = A.shape[0]
    g = _guard(wp)
    no = 2
    B = Bg * p_in
    sty_p = style.reshape(Bg // no, no, p_in * nz)
    A_p = _kron_eye(A, p_in)
    b2_p = jnp.tile(b2, (1, p_in))
    w_p = _kron_eye(w_mat, p_in)
    out = pl.pallas_call(
        functools.partial(_rgb_kernel, no=no),
        out_shape=jax.ShapeDtypeStruct((Bg, T, p_in * 3), jnp.float32),
        grid=(Bg // no,),
        in_specs=[
            pl.BlockSpec((1, no, p_in * nz), lambda i: (i, 0, 0)),
            pl.BlockSpec((no, T, cin_p), lambda i: (i, 0, 0)),
            pl.BlockSpec((p_in * nz, cin_p), lambda i: (0, 0)),
            pl.BlockSpec((1, cin_p), lambda i: (0, 0)),
            pl.BlockSpec((cin_p, p_in * 3), lambda i: (0, 0)),
        ],
        out_specs=pl.BlockSpec((no, T, p_in * 3), lambda i: (i, 0, 0)),
        compiler_params=pltpu.CompilerParams(
            dimension_semantics=("parallel",),
            vmem_limit_bytes=_VMEM_LIMIT,
        ),
    )(sty_p, x, A_p, b2_p, w_p)
    o = out[:, g:g + hp * wp, :].reshape(Bg, hp, wp, p_in, 3)
    o = jnp.transpose(o, (0, 3, 1, 2, 4)).reshape(B, hp, wp, 3)
    return o[:, y0:y0 + H, x0:x0 + W, :]


def kernel(styles, const,
           b0_A, b0_b2, b0_w_taps, b0_wsq, b0_noise_strength,
           b1_A, b1_b2, b1_w_taps, b1_wsq, b1_noise_strength,
           b2_A, b2_b2, b2_w_taps, b2_wsq, b2_noise_strength,
           b3_A, b3_b2, b3_w_taps, b3_wsq, b3_noise_strength,
           b4_A, b4_b2, b4_w_taps, b4_wsq, b4_noise_strength,
           b5_A, b5_b2, b5_w_taps, b5_wsq, b5_noise_strength,
           b6_A, b6_b2, b6_w_taps, b6_wsq, b6_noise_strength,
           b7_A, b7_b2, b7_w_taps, b7_wsq, b7_noise_strength,
           b8_A, b8_b2, b8_w_taps, b8_wsq, b8_noise_strength,
           rgb_A, rgb_b2, rgb_w_mat,
           noise0, noise1, noise2, noise3, noise4,
           noise5, noise6, noise7, noise8):
    blocks = [
        (b0_A, b0_b2, b0_w_taps, b0_wsq, b0_noise_strength),
        (b1_A, b1_b2, b1_w_taps, b1_wsq, b1_noise_strength),
        (b2_A, b2_b2, b2_w_taps, b2_wsq, b2_noise_strength),
        (b3_A, b3_b2, b3_w_taps, b3_wsq, b3_noise_strength),
        (b4_A, b4_b2, b4_w_taps, b4_wsq, b4_noise_strength),
        (b5_A, b5_b2, b5_w_taps, b5_wsq, b5_noise_strength),
        (b6_A, b6_b2, b6_w_taps, b6_wsq, b6_noise_strength),
        (b7_A, b7_b2, b7_w_taps, b7_wsq, b7_noise_strength),
        (b8_A, b8_b2, b8_w_taps, b8_wsq, b8_noise_strength),
    ]
    noises = [noise0, noise1, noise2, noise3, noise4,
              noise5, noise6, noise7, noise8]
    plan = [
        (1, 1, 16, True, False),
        (1, 1, 16, False, True),
        (1, 1, 16, True, False),
        (1, 1, 16, False, True),
        (1, 1, 16, True, False),
        (1, 2, 4, False, False),
        (2, 2, 1, True, False),
        (4, 1, 1, False, False),
        (4, 1, 1, False, False),
    ]
    B = styles.shape[0]
    nf = const.shape[-1]

    H = W = 4
    g0 = _guard(W + 2)
    cflat = jnp.pad(const, ((1, 1), (1, 1), (0, 0))).reshape((H + 2) * (W + 2), nf)
    cflat = jnp.pad(cflat, ((g0, g0), (0, 0)))
    x = jnp.broadcast_to(cflat[None], (B, cflat.shape[0], nf))
    geom = (H + 2, W + 2, 1, 1, H, W)

    for j, (A, b2, w_taps, wsq, ns) in enumerate(blocks):
        if _UPSAMPLE[j]:
            H, W = 2 * H, 2 * W
            geom = (2 * geom[0], 2 * geom[1], 2 * geom[2], 2 * geom[3], H, W)
        p_in, rg, no, up_out, trim_out = plan[j]
        x = _genblock_call(x, geom, styles[:, j, :], A, b2, w_taps, wsq, ns,
                           noises[j], p_in=p_in, rg=rg, no=no, up_out=up_out,
                           trim_out=trim_out)
        if trim_out:
            geom = (H + 2, W + 2, 1, 1, H, W)
    return _rgb_call(x, geom, styles[:, -1, :], rgb_A, rgb_b2, rgb_w_mat, p_in=4)
